# baseline (speedup 1.0000x reference)
import functools

import jax
import jax.numpy as jnp
from jax import lax
from jax.experimental import pallas as pl
from jax.experimental.pallas import tpu as pltpu
from jax.experimental.pallas import tpu_sc as plsc

_NUM_ENT = 1000000
_ITEM_IN_KG = 400000
_DIM = 32
_NN = 8


def _worker_count():
    info = plsc.get_sparse_core_info()
    return info.num_cores, info.num_subcores


@functools.lru_cache(maxsize=None)
def _make_sc_gather(n, chunk, n_gathers, widths, dtypes, idx_slots, n_idx):
    nc, ns = _worker_count()
    nw = nc * ns
    per_w = n // nw
    n_chunks = per_w // chunk
    assert per_w % chunk == 0 and n % nw == 0

    mesh = plsc.VectorSubcoreMesh(core_axis_name="c", subcore_axis_name="s")
    out_type = [jax.ShapeDtypeStruct((n, w), jnp.dtype(dt))
                for w, dt in zip(widths, dtypes)]
    scratch = ([pltpu.VMEM((chunk,), jnp.int32) for _ in range(n_idx)]
               + [pltpu.VMEM((chunk, w), jnp.dtype(dt))
                  for w, dt in zip(widths, dtypes)]
               + [pltpu.SemaphoreType.DMA])

    @functools.partial(
        pl.kernel, out_type=out_type, mesh=mesh, scratch_types=scratch,
        compiler_params=pltpu.CompilerParams(use_tc_tiling_on_sc=False))
    def sc_gather(*refs):
        idx_hbm = refs[:n_idx]
        tables = refs[n_idx:n_idx + n_gathers]
        outs = refs[n_idx + n_gathers:n_idx + 2 * n_gathers]
        idx_v = refs[n_idx + 2 * n_gathers:2 * n_idx + 2 * n_gathers]
        bufs = refs[2 * n_idx + 2 * n_gathers:-1]
        sem = refs[-1]
        wid = lax.axis_index("s") * nc + lax.axis_index("c")
        base = wid * per_w

        nsplit = 4
        sub = chunk // nsplit

        def body(i, carry):
            off = base + i * chunk
            for s in range(n_idx):
                pltpu.sync_copy(idx_hbm[s].at[pl.ds(off, chunk)], idx_v[s])
            for g in range(n_gathers):
                cps = [pltpu.async_copy(
                    tables[g].at[idx_v[idx_slots[g]].at[pl.ds(k * sub, sub)]],
                    bufs[g].at[pl.ds(k * sub, sub)], sem)
                    for k in range(nsplit)]
                for cp in cps:
                    cp.wait()
                pltpu.sync_copy(bufs[g], outs[g].at[pl.ds(off, chunk)])
            return carry

        lax.fori_loop(0, n_chunks, body, 0)

    return sc_gather


@functools.lru_cache(maxsize=None)
def _make_tc_dense(batch, bb):
    grid = (batch // bb,)

    def body(ue_ref, ev0_ref, ev1_ref, ev2_ref, r0_ref, r1_ref, rel_ref,
             w_ref, b_ref, out_ref):
        ue = ue_ref[...]
        rel = rel_ref[...]
        wm = w_ref[...]
        bias = b_ref[...]
        p = jnp.dot(ue, rel.T, preferred_element_type=jnp.float32)

        nrel = rel.shape[0]
        r0 = r0_ref[...]
        r1 = r1_ref[...]
        s0 = jnp.zeros((bb, _NN), jnp.float32)
        s1 = jnp.zeros((bb, 64), jnp.float32)
        for k in range(nrel):
            pk = p[:, k][:, None]
            s0 = s0 + jnp.where(r0 == k, pk, 0.0)
            s1 = s1 + jnp.where(r1 == k, pk, 0.0)
        a0 = jax.nn.softmax(s0, axis=-1)
        a1 = jax.nn.softmax(s1.reshape(bb, 8, 8), axis=-1)

        ev1 = ev1_ref[...].reshape(bb, 8, _DIM)
        ev2 = ev2_ref[...].reshape(bb, 8, 8, _DIM)

        agg1 = jnp.sum(a1[..., None] * ev2, axis=2)
        o1 = jax.nn.sigmoid(
            jnp.dot((ev1 + agg1).reshape(bb * 8, _DIM), wm.T,
                    preferred_element_type=jnp.float32) + bias)
        agg0 = jnp.sum(a0[..., None] * ev1, axis=1)
        o0 = jax.nn.sigmoid(
            jnp.dot(ev0_ref[...] + agg0, wm.T,
                    preferred_element_type=jnp.float32) + bias)
        aggf = jnp.sum(a0[..., None] * o1.reshape(bb, 8, _DIM), axis=1)
        of = jnp.tanh(jnp.dot(o0 + aggf, wm.T,
                              preferred_element_type=jnp.float32) + bias)
        out_ref[...] = jax.nn.sigmoid(jnp.sum(ue * of, axis=-1))

    return pl.pallas_call(
        body,
        grid=grid,
        in_specs=[
            pl.BlockSpec((bb, _DIM), lambda i: (i, 0)),
            pl.BlockSpec((bb, _DIM), lambda i: (i, 0)),
            pl.BlockSpec((bb * 8, _DIM), lambda i: (i, 0)),
            pl.BlockSpec((bb * 64, _DIM), lambda i: (i, 0)),
            pl.BlockSpec((bb, _NN), lambda i: (i, 0)),
            pl.BlockSpec((bb, 64), lambda i: (i, 0)),
            pl.BlockSpec((33, _DIM), lambda i: (0, 0)),
            pl.BlockSpec((_DIM, _DIM), lambda i: (0, 0)),
            pl.BlockSpec((1, _DIM), lambda i: (0, 0)),
        ],
        out_specs=pl.BlockSpec((bb,), lambda i: (i,)),
        out_shape=jax.ShapeDtypeStruct((batch,), jnp.float32),
    )


def _run_slab(u, v, adj_ent, adj_rel, item_emb, ent_emb, usr_emb, rel_emb,
              W, b):
    batch = u.shape[0]
    e0 = jnp.where(v >= _ITEM_IN_KG, _NUM_ENT, v).astype(jnp.int32)
    padj = adj_ent | (adj_rel << 20)

    g1 = _make_sc_gather(batch, min(512, batch // 32), 3,
                         (_NN, _DIM, _DIM),
                         ("int32", "float32", "float32"),
                         (0, 1, 2), 3)
    p1, ue, ev0 = g1(e0, u, v, padj, usr_emb, item_emb)
    r0 = p1 >> 20
    e1f = (p1 & 0xFFFFF).reshape(-1)

    g2 = _make_sc_gather(batch * _NN, 1024, 2,
                         (_NN, _DIM),
                         ("int32", "float32"),
                         (0, 0), 1)
    p2, ev1 = g2(e1f, padj, ent_emb)
    r1 = p2 >> 20
    e2 = p2 & 0xFFFFF

    e2f = e2.reshape(-1)
    g3 = _make_sc_gather(batch * _NN * _NN, 2048, 1,
                         (_DIM,), ("float32",), (0,), 1)
    (ev2,) = g3(e2f, ent_emb)

    dense = _make_tc_dense(batch, 256)
    return dense(ue, ev0, ev1, ev2, r0, r1.reshape(batch, 64),
                 rel_emb, W, b.reshape(1, _DIM))


def kernel(u, v, adj_ent, adj_rel, item_emb, ent_emb, usr_emb, rel_emb, W, b):
    batch = u.shape[0]
    n_slabs = 1
    sb = batch // n_slabs
    outs = []
    for s in range(n_slabs):
        sl = slice(s * sb, (s + 1) * sb)
        outs.append(_run_slab(u[sl], v[sl], adj_ent, adj_rel, item_emb,
                              ent_emb, usr_emb, rel_emb, W, b))
    return jnp.concatenate(outs)


# speedup vs baseline: 1.2847x; 1.2847x over previous
import functools

import jax
import jax.numpy as jnp
from jax import lax
from jax.experimental import pallas as pl
from jax.experimental.pallas import tpu as pltpu
from jax.experimental.pallas import tpu_sc as plsc

_NUM_ENT = 1000000
_ITEM_IN_KG = 400000
_DIM = 32
_NN = 8


def _worker_count():
    info = plsc.get_sparse_core_info()
    return info.num_cores, info.num_subcores


@functools.lru_cache(maxsize=None)
def _make_sc_gather(n, chunk, n_gathers, widths, dtypes, idx_slots, n_idx):
    nc, ns = _worker_count()
    nw = nc * ns
    per_w = n // nw
    n_chunks = per_w // chunk
    assert per_w % chunk == 0 and n % nw == 0

    mesh = plsc.VectorSubcoreMesh(core_axis_name="c", subcore_axis_name="s")
    out_type = [jax.ShapeDtypeStruct((n, w), jnp.dtype(dt))
                for w, dt in zip(widths, dtypes)]
    scratch = ([pltpu.VMEM((chunk,), jnp.int32) for _ in range(n_idx)]
               + [pltpu.VMEM((chunk, w), jnp.dtype(dt))
                  for w, dt in zip(widths, dtypes)]
               + [pltpu.SemaphoreType.DMA])

    @functools.partial(
        pl.kernel, out_type=out_type, mesh=mesh, scratch_types=scratch,
        compiler_params=pltpu.CompilerParams(use_tc_tiling_on_sc=False))
    def sc_gather(*refs):
        idx_hbm = refs[:n_idx]
        tables = refs[n_idx:n_idx + n_gathers]
        outs = refs[n_idx + n_gathers:n_idx + 2 * n_gathers]
        idx_v = refs[n_idx + 2 * n_gathers:2 * n_idx + 2 * n_gathers]
        bufs = refs[2 * n_idx + 2 * n_gathers:-1]
        sem = refs[-1]
        wid = lax.axis_index("s") * nc + lax.axis_index("c")
        base = wid * per_w

        def body(i, carry):
            off = base + i * chunk
            for s in range(n_idx):
                pltpu.sync_copy(idx_hbm[s].at[pl.ds(off, chunk)], idx_v[s])
            for g in range(n_gathers):
                pltpu.async_copy(tables[g].at[idx_v[idx_slots[g]]], bufs[g],
                                 sem).wait()
                pltpu.sync_copy(bufs[g], outs[g].at[pl.ds(off, chunk)])
            return carry

        lax.fori_loop(0, n_chunks, body, 0)

    return sc_gather


def _deinterleave(x):
    row = lax.broadcasted_iota(jnp.int32, (_DIM, _DIM), 0)
    col = lax.broadcasted_iota(jnp.int32, (_DIM, _DIM), 1)
    pcol = jnp.where(col < 16, 2 * col, 2 * (col - 16) + 1)
    pm = jnp.where(row == pcol, 1.0, 0.0)
    return jnp.dot(x, pm, preferred_element_type=jnp.float32)


@functools.lru_cache(maxsize=None)
def _make_tc_dense(batch, bb):
    grid = (batch // bb,)

    def body(ue_ref, ev0_ref, ev1_ref, ev2_ref, r0_ref, r1_ref, rel_ref,
             w_ref, wp_ref, b_ref, out_ref):
        ue = ue_ref[...]
        rel = rel_ref[...]
        wm = w_ref[...]
        wp = wp_ref[...]
        bias = b_ref[...]
        p = jnp.dot(ue, rel.T, preferred_element_type=jnp.float32)

        nrel = rel.shape[0]
        r0 = r0_ref[...]
        r1 = r1_ref[...]
        s0 = jnp.zeros((bb, _NN), jnp.float32)
        s1 = jnp.zeros((bb, 64), jnp.float32)
        for k in range(nrel):
            pk = p[:, k][:, None]
            s0 = s0 + jnp.where(r0 == k, pk, 0.0)
            s1 = s1 + jnp.where(r1 == k, pk, 0.0)
        a0 = jax.nn.softmax(s0, axis=-1)
        a1 = jax.nn.softmax(s1.reshape(bb, 8, 8), axis=-1)

        ev1 = _deinterleave(ev1_ref[...].reshape(bb * 8, _DIM))
        ev0 = _deinterleave(ev0_ref[...])

        pk2 = ev2_ref[...]
        ev2e = lax.bitcast_convert_type(pk2 << 16, jnp.float32)
        ev2o = lax.bitcast_convert_type(
            pk2 & jnp.int32(-65536), jnp.float32)
        lane = lax.broadcasted_iota(jnp.int32, (_NN, 128), 1) // 16
        nbr = lax.broadcasted_iota(jnp.int32, (_NN, 128), 0)
        expand = jnp.where(lane == nbr, 1.0, 0.0)
        w128 = jnp.dot(a1.reshape(bb * 8, _NN), expand,
                       preferred_element_type=jnp.float32)

        def red8(t):
            t = t + jnp.concatenate([t[:, 64:], t[:, :64]], axis=1)
            t = t + jnp.concatenate([t[:, 32:], t[:, :32]], axis=1)
            t = t + jnp.concatenate([t[:, 16:], t[:, :16]], axis=1)
            return t[:, :16]

        agg1 = jnp.concatenate([red8(ev2e * w128), red8(ev2o * w128)],
                               axis=-1)
        o1 = jax.nn.sigmoid(
            jnp.dot(ev1 + agg1, wp.T, preferred_element_type=jnp.float32)
            + bias)
        agg0 = jnp.sum(a0[..., None] * ev1.reshape(bb, 8, _DIM), axis=1)
        o0 = jax.nn.sigmoid(
            jnp.dot(ev0 + agg0, wp.T, preferred_element_type=jnp.float32)
            + bias)
        aggf = jnp.sum(a0[..., None] * o1.reshape(bb, 8, _DIM), axis=1)
        of = jnp.tanh(jnp.dot(o0 + aggf, wm.T,
                              preferred_element_type=jnp.float32) + bias)
        out_ref[...] = jax.nn.sigmoid(jnp.sum(ue * of, axis=-1))

    return pl.pallas_call(
        body,
        grid=grid,
        in_specs=[
            pl.BlockSpec((bb, _DIM), lambda i: (i, 0)),
            pl.BlockSpec((bb, _DIM), lambda i: (i, 0)),
            pl.BlockSpec((bb * 8, _DIM), lambda i: (i, 0)),
            pl.BlockSpec((bb * 8, 128), lambda i: (i, 0)),
            pl.BlockSpec((bb, _NN), lambda i: (i, 0)),
            pl.BlockSpec((bb, 64), lambda i: (i, 0)),
            pl.BlockSpec((33, _DIM), lambda i: (0, 0)),
            pl.BlockSpec((_DIM, _DIM), lambda i: (0, 0)),
            pl.BlockSpec((_DIM, _DIM), lambda i: (0, 0)),
            pl.BlockSpec((1, _DIM), lambda i: (0, 0)),
        ],
        out_specs=pl.BlockSpec((bb,), lambda i: (i,)),
        out_shape=jax.ShapeDtypeStruct((batch,), jnp.float32),
    )


def kernel(u, v, adj_ent, adj_rel, item_emb, ent_emb, usr_emb, rel_emb, W, b):
    batch = u.shape[0]
    e0 = jnp.where(v >= _ITEM_IN_KG, _NUM_ENT, v).astype(jnp.int32)
    ne = ent_emb.shape[0]
    ent_pk = lax.bitcast_convert_type(
        ent_emb.astype(jnp.bfloat16).reshape(ne, _DIM // 2, 2), jnp.int32)

    g1 = _make_sc_gather(batch, min(512, batch // 32), 4,
                         (_NN, _NN, _DIM, _DIM),
                         ("int32", "int32", "float32", "float32"),
                         (0, 0, 1, 2), 3)
    e1, r0, ue, ev0 = g1(e0, u, v, adj_ent, adj_rel, usr_emb, item_emb)

    e1f = e1.reshape(-1)
    g2 = _make_sc_gather(batch * _NN, 1024, 3,
                         (_NN, _NN, _DIM),
                         ("int32", "int32", "float32"),
                         (0, 0, 0), 1)
    e2, r1, ev1 = g2(e1f, adj_ent, adj_rel, ent_emb)

    e2f = e2.reshape(-1)
    g3 = _make_sc_gather(batch * _NN * _NN, 2048, 1,
                         (_DIM // 2,), ("int32",), (0,), 1)
    (ev2,) = g3(e2f, ent_pk)

    perm = [2 * i for i in range(16)] + [2 * i + 1 for i in range(16)]
    wp = W[:, jnp.array(perm, dtype=jnp.int32)]
    dense = _make_tc_dense(batch, 256)
    return dense(ue, ev0, ev1, ev2.reshape(batch * 8, 128), r0,
                 r1.reshape(batch, 64), rel_emb, W, wp, b.reshape(1, _DIM))
